# Initial kernel scaffold; baseline (speedup 1.0000x reference)
import jax
import jax.numpy as jnp
from jax.experimental import pallas as pl


def kernel(in_nodes_features, edge_index, linear_proj, scoring_fn_source, scoring_fn_target):
    raise NotImplementedError("write your pallas kernel here")


# speedup vs baseline: 6.0333x; 6.0333x over previous
import functools

import jax
import jax.numpy as jnp
from jax import lax
from jax.experimental import pallas as pl
from jax.experimental.pallas import tpu as pltpu
from jax.experimental.pallas import tpu_sc as plsc

N_NODES = 10000
N_EDGES = 320000
D_IN = 128
NH = 8
F_OUT = 16

NPAD = 10112
EPAD = 327680
EROWS = EPAD // 128
CHUNK = 1024
NCHUNK = EPAD // (32 * CHUNK)

_mesh = plsc.VectorSubcoreMesh(
    core_axis_name="c", subcore_axis_name="s", num_cores=2, num_subcores=16
)


def _scores_body(x_ref, w_ref, asrc_ref, atrg_ref, ss_ref, st_ref):
    x = x_ref[...]
    w = w_ref[...]
    proj = jnp.dot(x, w, preferred_element_type=jnp.float32)
    col = lax.broadcasted_iota(jnp.int32, (D_IN, NH), 0)
    hd = lax.broadcasted_iota(jnp.int32, (D_IN, NH), 1)
    g = (col // F_OUT == hd).astype(jnp.float32)
    ss_ref[...] = jnp.dot(proj * asrc_ref[...], g, preferred_element_type=jnp.float32)
    st_ref[...] = jnp.dot(proj * atrg_ref[...], g, preferred_element_type=jnp.float32)


def _scores(x_pad, w, asrc, atrg):
    return pl.pallas_call(
        _scores_body,
        out_shape=[
            jax.ShapeDtypeStruct((NPAD, NH), jnp.float32),
            jax.ShapeDtypeStruct((NPAD, NH), jnp.float32),
        ],
    )(x_pad, w, asrc, atrg)


def _p1_body(src_ref, trg_ref, ss_ref, st_ref, z_ref, exp_ref, part_ref,
             idx_s, idx_t, rows_s, rows_t, exp_v, stage_v, denom_sp, sem_g, sem_w):
    cid = lax.axis_index("c")
    sid = lax.axis_index("s")
    wid = sid * 2 + cid

    zrows = pl.ds(sid * (NPAD // 16), NPAD // 16)
    pltpu.sync_copy(z_ref.at[zrows], stage_v)
    pltpu.sync_copy(stage_v, denom_sp.at[zrows])
    plsc.subcore_barrier()

    lane = lax.iota(jnp.int32, 16)
    prow = jnp.right_shift(lane, 3)
    pcol = jnp.bitwise_and(lane, 7)

    def chunk(c, carry):
        r0 = wid * (NCHUNK * 8) + c * 8
        pltpu.sync_copy(src_ref.at[pl.ds(r0, 8)], idx_s)
        pltpu.sync_copy(trg_ref.at[pl.ds(r0, 8)], idx_t)
        cps = []
        for j in range(8):
            cps.append(pltpu.async_copy(
                ss_ref.at[idx_s.at[j]], rows_s.at[pl.ds(j * 128, 128)], sem_g))
            cps.append(pltpu.async_copy(
                st_ref.at[idx_t.at[j]], rows_t.at[pl.ds(j * 128, 128)], sem_g))
        for cp in cps:
            cp.wait()

        def vec(i, acc):
            rv = prow + 2 * i
            a = plsc.load_gather(rows_s, [rv, pcol])
            b = plsc.load_gather(rows_t, [rv, pcol])
            x = a + b
            wv = jnp.exp(jnp.maximum(x, x * 0.2))
            plsc.store_scatter(exp_v, [rv, pcol], wv)
            return acc

        lax.fori_loop(0, CHUNK // 2, vec, 0)

        ebase = wid * (NCHUNK * CHUNK) + c * CHUNK
        wcp = pltpu.async_copy(exp_v, exp_ref.at[pl.ds(ebase, CHUNK)], sem_w)
        for j in range(8):
            pltpu.sync_copy(
                exp_v.at[pl.ds(j * 128, 128)], denom_sp.at[idx_t.at[j]],
                add=True)
        wcp.wait()
        return carry

    lax.fori_loop(0, NCHUNK, chunk, 0)

    plsc.subcore_barrier()
    pltpu.sync_copy(denom_sp.at[zrows], stage_v)
    pltpu.sync_copy(stage_v, part_ref.at[cid, zrows])


def _pass1(src2d, trg2d, ss, st, zeros):
    f = pl.kernel(
        _p1_body,
        out_type=[
            jax.ShapeDtypeStruct((EPAD, NH), jnp.float32),
            jax.ShapeDtypeStruct((2, NPAD, NH), jnp.float32),
        ],
        mesh=_mesh,
        compiler_params=pltpu.CompilerParams(
            needs_layout_passes=False, use_tc_tiling_on_sc=False),
        scratch_types=[
            pltpu.VMEM((8, 128), jnp.int32),
            pltpu.VMEM((8, 128), jnp.int32),
            pltpu.VMEM((CHUNK, NH), jnp.float32),
            pltpu.VMEM((CHUNK, NH), jnp.float32),
            pltpu.VMEM((CHUNK, NH), jnp.float32),
            pltpu.VMEM((NPAD // 16, NH), jnp.float32),
            pltpu.VMEM_SHARED((NPAD, NH), jnp.float32),
            pltpu.SemaphoreType.DMA,
            pltpu.SemaphoreType.DMA,
        ],
    )
    return f(src2d, trg2d, ss, st, zeros)


def _p2_body(trg_ref, exp_ref, p0_ref, p1_ref, out_ref,
             idx_t, ev, r0v, r1v, ov, sem_g):
    cid = lax.axis_index("c")
    sid = lax.axis_index("s")
    wid = sid * 2 + cid

    lane = lax.iota(jnp.int32, 16)
    prow = jnp.right_shift(lane, 3)
    pcol = jnp.bitwise_and(lane, 7)

    def chunk(c, carry):
        r0 = wid * (NCHUNK * 8) + c * 8
        pltpu.sync_copy(trg_ref.at[pl.ds(r0, 8)], idx_t)
        cps = []
        for j in range(8):
            cps.append(pltpu.async_copy(
                p0_ref.at[idx_t.at[j]], r0v.at[pl.ds(j * 128, 128)], sem_g))
            cps.append(pltpu.async_copy(
                p1_ref.at[idx_t.at[j]], r1v.at[pl.ds(j * 128, 128)], sem_g))
        ebase = wid * (NCHUNK * CHUNK) + c * CHUNK
        cps.append(pltpu.async_copy(exp_ref.at[pl.ds(ebase, CHUNK)], ev, sem_g))
        for cp in cps:
            cp.wait()

        def vec(i, acc):
            rv = prow + 2 * i
            e = plsc.load_gather(ev, [rv, pcol])
            a = plsc.load_gather(r0v, [rv, pcol])
            b = plsc.load_gather(r1v, [rv, pcol])
            o = e / (a + b + 1e-16)
            plsc.store_scatter(ov, [rv, pcol], o)
            return acc

        lax.fori_loop(0, CHUNK // 2, vec, 0)
        pltpu.sync_copy(ov, out_ref.at[pl.ds(ebase, CHUNK)])
        return carry

    lax.fori_loop(0, NCHUNK, chunk, 0)


def _pass2(trg2d, exp_e, p0, p1):
    f = pl.kernel(
        _p2_body,
        out_type=jax.ShapeDtypeStruct((EPAD, NH), jnp.float32),
        mesh=_mesh,
        compiler_params=pltpu.CompilerParams(
            needs_layout_passes=False, use_tc_tiling_on_sc=False),
        scratch_types=[
            pltpu.VMEM((8, 128), jnp.int32),
            pltpu.VMEM((CHUNK, NH), jnp.float32),
            pltpu.VMEM((CHUNK, NH), jnp.float32),
            pltpu.VMEM((CHUNK, NH), jnp.float32),
            pltpu.VMEM((CHUNK, NH), jnp.float32),
            pltpu.SemaphoreType.DMA,
        ],
    )
    return f(trg2d, exp_e, p0, p1)


def kernel(in_nodes_features, edge_index, linear_proj, scoring_fn_source,
           scoring_fn_target):
    x_pad = jnp.pad(in_nodes_features, ((0, NPAD - N_NODES), (0, 0)))
    asrc = scoring_fn_source.reshape(1, NH * F_OUT)
    atrg = scoring_fn_target.reshape(1, NH * F_OUT)
    ss, st = _scores(x_pad, linear_proj, asrc, atrg)

    src = jnp.pad(edge_index[0], (0, EPAD - N_EDGES)).reshape(EROWS, 128)
    trg = jnp.pad(edge_index[1], (0, EPAD - N_EDGES),
                  constant_values=N_NODES).reshape(EROWS, 128)
    zeros = jnp.zeros((NPAD, NH), jnp.float32)

    exp_e, parts = _pass1(src, trg, ss, st, zeros)
    att = _pass2(trg, exp_e, parts[0], parts[1])
    return att[:N_EDGES].reshape(N_EDGES, NH, 1)


# speedup vs baseline: 8.1380x; 1.3489x over previous
import functools

import jax
import jax.numpy as jnp
from jax import lax
from jax.experimental import pallas as pl
from jax.experimental.pallas import tpu as pltpu
from jax.experimental.pallas import tpu_sc as plsc

N_NODES = 10000
N_EDGES = 320000
D_IN = 128
NH = 8
F_OUT = 16

NPAD = 10112
CHUNK = 1000
NCHUNK = N_EDGES // (32 * CHUNK)

_mesh = plsc.VectorSubcoreMesh(
    core_axis_name="c", subcore_axis_name="s", num_cores=2, num_subcores=16
)


def _scores_body(x_ref, w_ref, asrc_ref, atrg_ref, ss_ref, st_ref):
    x = x_ref[...]
    w = w_ref[...]
    proj = jnp.dot(x, w, preferred_element_type=jnp.float32)
    col = lax.broadcasted_iota(jnp.int32, (D_IN, NH), 0)
    hd = lax.broadcasted_iota(jnp.int32, (D_IN, NH), 1)
    g = (col // F_OUT == hd).astype(jnp.float32)
    ss_ref[...] = jnp.dot(proj * asrc_ref[...], g, preferred_element_type=jnp.float32)
    st_ref[...] = jnp.dot(proj * atrg_ref[...], g, preferred_element_type=jnp.float32)


def _scores(x_pad, w, asrc, atrg):
    return pl.pallas_call(
        _scores_body,
        out_shape=[
            jax.ShapeDtypeStruct((NPAD, NH), jnp.float32),
            jax.ShapeDtypeStruct((NPAD, NH), jnp.float32),
        ],
    )(x_pad, w, asrc, atrg)


def _p1_body(src_ref, trg_ref, ss_ref, st_ref, z_ref, exp_ref, part_ref,
             idx_s, idx_t, rows_s, rows_t, exp_v, stage_v, denom_sp, sem_g, sem_w):
    cid = lax.axis_index("c")
    sid = lax.axis_index("s")
    wid = sid * 2 + cid

    zrows = pl.ds(sid * (NPAD // 16), NPAD // 16)
    pltpu.sync_copy(z_ref.at[zrows], stage_v)
    pltpu.sync_copy(stage_v, denom_sp.at[zrows])
    plsc.subcore_barrier()

    lane = lax.iota(jnp.int32, 16)
    prow = jnp.right_shift(lane, 3)
    pcol = jnp.bitwise_and(lane, 7)

    def chunk(c, carry):
        ebase = wid * (NCHUNK * CHUNK) + c * CHUNK
        pltpu.sync_copy(src_ref.at[pl.ds(ebase, CHUNK)], idx_s)
        pltpu.sync_copy(trg_ref.at[pl.ds(ebase, CHUNK)], idx_t)
        g1 = pltpu.async_copy(ss_ref.at[idx_s], rows_s, sem_g)
        g2 = pltpu.async_copy(st_ref.at[idx_t], rows_t, sem_g)
        g1.wait()
        g2.wait()

        def vec(i, acc):
            rv = prow + 2 * i
            a = plsc.load_gather(rows_s, [rv, pcol])
            b = plsc.load_gather(rows_t, [rv, pcol])
            x = a + b
            wv = jnp.exp(jnp.maximum(x, x * 0.2))
            plsc.store_scatter(exp_v, [rv, pcol], wv)
            return acc

        lax.fori_loop(0, CHUNK // 2, vec, 0, unroll=8)

        wcp = pltpu.async_copy(exp_v, exp_ref.at[pl.ds(ebase, CHUNK)], sem_w)
        pltpu.sync_copy(exp_v, denom_sp.at[idx_t], add=True)
        wcp.wait()
        return carry

    lax.fori_loop(0, NCHUNK, chunk, 0)

    plsc.subcore_barrier()
    pltpu.sync_copy(denom_sp.at[zrows], stage_v)
    pltpu.sync_copy(stage_v, part_ref.at[cid, zrows])


def _pass1(src2d, trg2d, ss, st, zeros):
    f = pl.kernel(
        _p1_body,
        out_type=[
            jax.ShapeDtypeStruct((N_EDGES, NH), jnp.float32),
            jax.ShapeDtypeStruct((2, NPAD, NH), jnp.float32),
        ],
        mesh=_mesh,
        compiler_params=pltpu.CompilerParams(
            needs_layout_passes=False, use_tc_tiling_on_sc=False),
        scratch_types=[
            pltpu.VMEM((CHUNK,), jnp.int32),
            pltpu.VMEM((CHUNK,), jnp.int32),
            pltpu.VMEM((CHUNK, NH), jnp.float32),
            pltpu.VMEM((CHUNK, NH), jnp.float32),
            pltpu.VMEM((CHUNK, NH), jnp.float32),
            pltpu.VMEM((NPAD // 16, NH), jnp.float32),
            pltpu.VMEM_SHARED((NPAD, NH), jnp.float32),
            pltpu.SemaphoreType.DMA,
            pltpu.SemaphoreType.DMA,
        ],
    )
    return f(src2d, trg2d, ss, st, zeros)


def _p2_body(trg_ref, exp_ref, p0_ref, p1_ref, out_ref,
             idx_t, ev, r0v, r1v, ov, sem_g):
    cid = lax.axis_index("c")
    sid = lax.axis_index("s")
    wid = sid * 2 + cid

    lane = lax.iota(jnp.int32, 16)
    prow = jnp.right_shift(lane, 3)
    pcol = jnp.bitwise_and(lane, 7)

    def chunk(c, carry):
        ebase = wid * (NCHUNK * CHUNK) + c * CHUNK
        pltpu.sync_copy(trg_ref.at[pl.ds(ebase, CHUNK)], idx_t)
        cps = []
        cps.append(pltpu.async_copy(p0_ref.at[idx_t], r0v, sem_g))
        cps.append(pltpu.async_copy(p1_ref.at[idx_t], r1v, sem_g))
        cps.append(pltpu.async_copy(exp_ref.at[pl.ds(ebase, CHUNK)], ev, sem_g))
        for cp in cps:
            cp.wait()

        def vec(i, acc):
            rv = prow + 2 * i
            e = plsc.load_gather(ev, [rv, pcol])
            a = plsc.load_gather(r0v, [rv, pcol])
            b = plsc.load_gather(r1v, [rv, pcol])
            o = e / (a + b + 1e-16)
            plsc.store_scatter(ov, [rv, pcol], o)
            return acc

        lax.fori_loop(0, CHUNK // 2, vec, 0, unroll=8)
        pltpu.sync_copy(ov, out_ref.at[pl.ds(ebase, CHUNK)])
        return carry

    lax.fori_loop(0, NCHUNK, chunk, 0)


def _pass2(trg2d, exp_e, p0, p1):
    f = pl.kernel(
        _p2_body,
        out_type=jax.ShapeDtypeStruct((N_EDGES, NH), jnp.float32),
        mesh=_mesh,
        compiler_params=pltpu.CompilerParams(
            needs_layout_passes=False, use_tc_tiling_on_sc=False),
        scratch_types=[
            pltpu.VMEM((CHUNK,), jnp.int32),
            pltpu.VMEM((CHUNK, NH), jnp.float32),
            pltpu.VMEM((CHUNK, NH), jnp.float32),
            pltpu.VMEM((CHUNK, NH), jnp.float32),
            pltpu.VMEM((CHUNK, NH), jnp.float32),
            pltpu.SemaphoreType.DMA,
        ],
    )
    return f(trg2d, exp_e, p0, p1)


def kernel(in_nodes_features, edge_index, linear_proj, scoring_fn_source,
           scoring_fn_target):
    x_pad = jnp.pad(in_nodes_features, ((0, NPAD - N_NODES), (0, 0)))
    asrc = scoring_fn_source.reshape(1, NH * F_OUT)
    atrg = scoring_fn_target.reshape(1, NH * F_OUT)
    ss, st = _scores(x_pad, linear_proj, asrc, atrg)

    src = edge_index[0]
    trg = edge_index[1]
    zeros = jnp.zeros((NPAD, NH), jnp.float32)

    exp_e, parts = _pass1(src, trg, ss, st, zeros)
    att = _pass2(trg, exp_e, parts[0], parts[1])
    return att.reshape(N_EDGES, NH, 1)


# speedup vs baseline: 9.2560x; 1.1374x over previous
import jax
import jax.numpy as jnp
from jax import lax
from jax.experimental import pallas as pl
from jax.experimental.pallas import tpu as pltpu
from jax.experimental.pallas import tpu_sc as plsc

N_NODES = 10000
N_EDGES = 320000
D_IN = 128
NH = 8
F_OUT = 16

NPAD = 10112
CHUNK = 1000
NCHUNK = N_EDGES // (32 * CHUNK)

_mesh = plsc.VectorSubcoreMesh(
    core_axis_name="c", subcore_axis_name="s", num_cores=2, num_subcores=16
)
_params = pltpu.CompilerParams(
    needs_layout_passes=False, use_tc_tiling_on_sc=False)


def _scores_body(x_ref, w_ref, asrc_ref, atrg_ref, ss_ref, st_ref):
    x = x_ref[...]
    w = w_ref[...]
    proj = jnp.dot(x, w, preferred_element_type=jnp.float32)
    col = lax.broadcasted_iota(jnp.int32, (D_IN, NH), 0)
    hd = lax.broadcasted_iota(jnp.int32, (D_IN, NH), 1)
    g = (col // F_OUT == hd).astype(jnp.float32)
    ss_ref[...] = jnp.dot(proj * asrc_ref[...], g, preferred_element_type=jnp.float32)
    st_ref[...] = jnp.dot(proj * atrg_ref[...], g, preferred_element_type=jnp.float32)


def _scores(x_pad, w, asrc, atrg):
    return pl.pallas_call(
        _scores_body,
        out_shape=[
            jax.ShapeDtypeStruct((NPAD, NH), jnp.float32),
            jax.ShapeDtypeStruct((NPAD, NH), jnp.float32),
        ],
    )(x_pad, w, asrc, atrg)


def _lane_patterns():
    lane = lax.iota(jnp.int32, 16)
    return jnp.right_shift(lane, 3), jnp.bitwise_and(lane, 7)


def _p1_body(src_ref, trg_ref, ss_ref, st_ref, z_ref, exp_ref, part_ref,
             idx_s0, idx_t0, rows_s0, rows_t0, exp_v0,
             idx_s1, idx_t1, rows_s1, rows_t1, exp_v1,
             stage_v, denom_sp, sem_i, sem_g, sem_w):
    cid = lax.axis_index("c")
    sid = lax.axis_index("s")
    wid = sid * 2 + cid

    zrows = pl.ds(sid * (NPAD // 16), NPAD // 16)
    pltpu.sync_copy(z_ref.at[zrows], stage_v)
    pltpu.sync_copy(stage_v, denom_sp.at[zrows])
    plsc.subcore_barrier()

    prow, pcol = _lane_patterns()
    bufs = ((idx_s0, idx_t0, rows_s0, rows_t0, exp_v0),
            (idx_s1, idx_t1, rows_s1, rows_t1, exp_v1))

    def eb(c):
        return wid * (NCHUNK * CHUNK) + c * CHUNK

    Is, Gs = {}, {}

    def start_i(c):
        bs = bufs[c & 1]
        Is[c] = (
            pltpu.async_copy(src_ref.at[pl.ds(eb(c), CHUNK)], bs[0], sem_i),
            pltpu.async_copy(trg_ref.at[pl.ds(eb(c), CHUNK)], bs[1], sem_i),
        )

    def start_g(c):
        bs = bufs[c & 1]
        Gs[c] = (
            pltpu.async_copy(ss_ref.at[bs[0]], bs[2], sem_g),
            pltpu.async_copy(st_ref.at[bs[1]], bs[3], sem_g),
        )

    start_i(0)
    start_i(1)
    for cp in Is[0]:
        cp.wait()
    start_g(0)

    for c in range(NCHUNK):
        bs = bufs[c & 1]
        for cp in Gs[c]:
            cp.wait()
        if c + 1 < NCHUNK:
            for cp in Is[c + 1]:
                cp.wait()
            start_g(c + 1)

        rs, rt, ev = bs[2], bs[3], bs[4]

        def vec(i, acc):
            rv = prow + 2 * i
            a = plsc.load_gather(rs, [rv, pcol])
            b = plsc.load_gather(rt, [rv, pcol])
            x = a + b
            wv = jnp.exp(jnp.maximum(x, x * 0.2))
            plsc.store_scatter(ev, [rv, pcol], wv)
            return acc

        lax.fori_loop(0, CHUNK // 2, vec, 0, unroll=8)

        wcp = pltpu.async_copy(ev, exp_ref.at[pl.ds(eb(c), CHUNK)], sem_w)
        pltpu.sync_copy(ev, denom_sp.at[bs[1]], add=True)
        if c + 2 < NCHUNK:
            start_i(c + 2)
        wcp.wait()

    plsc.subcore_barrier()
    pltpu.sync_copy(denom_sp.at[zrows], stage_v)
    pltpu.sync_copy(stage_v, part_ref.at[cid, zrows])


def _pass1(src1d, trg1d, ss, st, zeros):
    f = pl.kernel(
        _p1_body,
        out_type=[
            jax.ShapeDtypeStruct((N_EDGES, NH), jnp.float32),
            jax.ShapeDtypeStruct((2, NPAD, NH), jnp.float32),
        ],
        mesh=_mesh,
        compiler_params=_params,
        scratch_types=[
            pltpu.VMEM((CHUNK,), jnp.int32),
            pltpu.VMEM((CHUNK,), jnp.int32),
            pltpu.VMEM((CHUNK, NH), jnp.float32),
            pltpu.VMEM((CHUNK, NH), jnp.float32),
            pltpu.VMEM((CHUNK, NH), jnp.float32),
            pltpu.VMEM((CHUNK,), jnp.int32),
            pltpu.VMEM((CHUNK,), jnp.int32),
            pltpu.VMEM((CHUNK, NH), jnp.float32),
            pltpu.VMEM((CHUNK, NH), jnp.float32),
            pltpu.VMEM((CHUNK, NH), jnp.float32),
            pltpu.VMEM((NPAD // 16, NH), jnp.float32),
            pltpu.VMEM_SHARED((NPAD, NH), jnp.float32),
            pltpu.SemaphoreType.DMA,
            pltpu.SemaphoreType.DMA,
            pltpu.SemaphoreType.DMA,
        ],
    )
    return f(src1d, trg1d, ss, st, zeros)


def _p2_body(trg_ref, exp_ref, p0_ref, p1_ref, out_ref,
             idx_t0, ev0, r0v0, r1v0, ov0,
             idx_t1, ev1, r0v1, r1v1, ov1,
             sem_i, sem_e, sem_g, sem_w):
    cid = lax.axis_index("c")
    sid = lax.axis_index("s")
    wid = sid * 2 + cid

    prow, pcol = _lane_patterns()
    bufs = ((idx_t0, ev0, r0v0, r1v0, ov0),
            (idx_t1, ev1, r0v1, r1v1, ov1))

    def eb(c):
        return wid * (NCHUNK * CHUNK) + c * CHUNK

    Is, Es, Gs = {}, {}, {}

    def start_i(c):
        bs = bufs[c & 1]
        Is[c] = pltpu.async_copy(trg_ref.at[pl.ds(eb(c), CHUNK)], bs[0], sem_i)
        Es[c] = pltpu.async_copy(exp_ref.at[pl.ds(eb(c), CHUNK)], bs[1], sem_e)

    def start_g(c):
        bs = bufs[c & 1]
        Gs[c] = (
            pltpu.async_copy(p0_ref.at[bs[0]], bs[2], sem_g),
            pltpu.async_copy(p1_ref.at[bs[0]], bs[3], sem_g),
        )

    start_i(0)
    start_i(1)
    Is[0].wait()
    start_g(0)

    for c in range(NCHUNK):
        bs = bufs[c & 1]
        for cp in Gs[c]:
            cp.wait()
        Es[c].wait()
        if c + 1 < NCHUNK:
            Is[c + 1].wait()
            start_g(c + 1)

        ev, r0v, r1v, ov = bs[1], bs[2], bs[3], bs[4]

        def vec(i, acc):
            rv = prow + 2 * i
            e = plsc.load_gather(ev, [rv, pcol])
            a = plsc.load_gather(r0v, [rv, pcol])
            b = plsc.load_gather(r1v, [rv, pcol])
            o = e / (a + b + 1e-16)
            plsc.store_scatter(ov, [rv, pcol], o)
            return acc

        lax.fori_loop(0, CHUNK // 2, vec, 0, unroll=8)

        wcp = pltpu.async_copy(ov, out_ref.at[pl.ds(eb(c), CHUNK)], sem_w)
        if c + 2 < NCHUNK:
            start_i(c + 2)
        wcp.wait()


def _pass2(trg1d, exp_e, p0, p1):
    f = pl.kernel(
        _p2_body,
        out_type=jax.ShapeDtypeStruct((N_EDGES, NH), jnp.float32),
        mesh=_mesh,
        compiler_params=_params,
        scratch_types=[
            pltpu.VMEM((CHUNK,), jnp.int32),
            pltpu.VMEM((CHUNK, NH), jnp.float32),
            pltpu.VMEM((CHUNK, NH), jnp.float32),
            pltpu.VMEM((CHUNK, NH), jnp.float32),
            pltpu.VMEM((CHUNK, NH), jnp.float32),
            pltpu.VMEM((CHUNK,), jnp.int32),
            pltpu.VMEM((CHUNK, NH), jnp.float32),
            pltpu.VMEM((CHUNK, NH), jnp.float32),
            pltpu.VMEM((CHUNK, NH), jnp.float32),
            pltpu.VMEM((CHUNK, NH), jnp.float32),
            pltpu.SemaphoreType.DMA,
            pltpu.SemaphoreType.DMA,
            pltpu.SemaphoreType.DMA,
            pltpu.SemaphoreType.DMA,
        ],
    )
    return f(trg1d, exp_e, p0, p1)


def kernel(in_nodes_features, edge_index, linear_proj, scoring_fn_source,
           scoring_fn_target):
    x_pad = jnp.pad(in_nodes_features, ((0, NPAD - N_NODES), (0, 0)))
    asrc = scoring_fn_source.reshape(1, NH * F_OUT)
    atrg = scoring_fn_target.reshape(1, NH * F_OUT)
    ss, st = _scores(x_pad, linear_proj, asrc, atrg)

    src = edge_index[0]
    trg = edge_index[1]
    zeros = jnp.zeros((NPAD, NH), jnp.float32)

    exp_e, parts = _pass1(src, trg, ss, st, zeros)
    att = _pass2(trg, exp_e, parts[0], parts[1])
    return att.reshape(N_EDGES, NH, 1)


# speedup vs baseline: 15.1522x; 1.6370x over previous
import jax
import jax.numpy as jnp
from jax import lax
from jax.experimental import pallas as pl
from jax.experimental.pallas import tpu as pltpu
from jax.experimental.pallas import tpu_sc as plsc

N_NODES = 10000
N_EDGES = 320000
D_IN = 128
NH = 8
F_OUT = 16

NPAD = 10112
CHUNK = 1000
NCHUNK = N_EDGES // (32 * CHUNK)

_mesh = plsc.VectorSubcoreMesh(
    core_axis_name="c", subcore_axis_name="s", num_cores=2, num_subcores=16
)
_params = pltpu.CompilerParams(
    needs_layout_passes=False, use_tc_tiling_on_sc=False)


def _scores_body(x_ref, w_ref, asrc_ref, atrg_ref, ss_ref, st_ref):
    x = x_ref[...]
    w = w_ref[...]
    proj = jnp.dot(x, w, preferred_element_type=jnp.float32)
    col = lax.broadcasted_iota(jnp.int32, (D_IN, NH), 0)
    hd = lax.broadcasted_iota(jnp.int32, (D_IN, NH), 1)
    g = (col // F_OUT == hd).astype(jnp.float32)
    ss_ref[...] = jnp.dot(proj * asrc_ref[...], g, preferred_element_type=jnp.float32)
    st_ref[...] = jnp.dot(proj * atrg_ref[...], g, preferred_element_type=jnp.float32)


def _scores(x_pad, w, asrc, atrg):
    return pl.pallas_call(
        _scores_body,
        out_shape=[
            jax.ShapeDtypeStruct((NPAD, NH), jnp.float32),
            jax.ShapeDtypeStruct((NPAD, NH), jnp.float32),
        ],
    )(x_pad, w, asrc, atrg)


def _lane_patterns():
    lane = lax.iota(jnp.int32, 16)
    return jnp.right_shift(lane, 3), jnp.bitwise_and(lane, 7)


def _p1_body(src_ref, trg_ref, ss_ref, st_ref, z_ref, exp_ref, part_ref,
             idx_s0, idx_t0, rows_s0, rows_t0, exp_v0,
             idx_s1, idx_t1, rows_s1, rows_t1, exp_v1,
             stage_v, denom_sp, sem_i, sem_g, sem_w):
    cid = lax.axis_index("c")
    sid = lax.axis_index("s")
    wid = sid * 2 + cid

    zrows = pl.ds(sid * (NPAD // 16), NPAD // 16)
    pltpu.sync_copy(z_ref.at[zrows], stage_v)
    pltpu.sync_copy(stage_v, denom_sp.at[zrows])
    plsc.subcore_barrier()

    prow, pcol = _lane_patterns()
    bufs = ((idx_s0, idx_t0, rows_s0, rows_t0, exp_v0),
            (idx_s1, idx_t1, rows_s1, rows_t1, exp_v1))

    def eb(c):
        return wid * (NCHUNK * CHUNK) + c * CHUNK

    Is, Gs = {}, {}

    def start_i(c):
        bs = bufs[c & 1]
        Is[c] = (
            pltpu.async_copy(src_ref.at[pl.ds(eb(c), CHUNK)], bs[0], sem_i),
            pltpu.async_copy(trg_ref.at[pl.ds(eb(c), CHUNK)], bs[1], sem_i),
        )

    def start_g(c):
        bs = bufs[c & 1]
        Gs[c] = (
            pltpu.async_copy(ss_ref.at[bs[0]], bs[2], sem_g),
            pltpu.async_copy(st_ref.at[bs[1]], bs[3], sem_g),
        )

    start_i(0)
    start_i(1)
    for cp in Is[0]:
        cp.wait()
    start_g(0)

    for c in range(NCHUNK):
        bs = bufs[c & 1]
        for cp in Gs[c]:
            cp.wait()
        if c + 1 < NCHUNK:
            for cp in Is[c + 1]:
                cp.wait()
            start_g(c + 1)

        rs, rt, ev = bs[2], bs[3], bs[4]

        def vec(i, acc):
            rv = prow + 2 * i
            a = plsc.load_gather(rs, [rv, pcol])
            b = plsc.load_gather(rt, [rv, pcol])
            x = a + b
            wv = jnp.exp(jnp.maximum(x, x * 0.2))
            plsc.store_scatter(ev, [rv, pcol], wv)
            return acc

        lax.fori_loop(0, CHUNK // 2, vec, 0, unroll=8)

        wcp = pltpu.async_copy(ev, exp_ref.at[pl.ds(eb(c), CHUNK)], sem_w)
        pltpu.sync_copy(ev, denom_sp.at[bs[1]], add=True)
        if c + 2 < NCHUNK:
            start_i(c + 2)
        wcp.wait()

    plsc.subcore_barrier()
    pltpu.sync_copy(denom_sp.at[zrows], stage_v)
    pltpu.sync_copy(stage_v, part_ref.at[cid, zrows])


def _pass1(src1d, trg1d, ss, st, zeros):
    f = pl.kernel(
        _p1_body,
        out_type=[
            jax.ShapeDtypeStruct((N_EDGES, NH), jnp.float32),
            jax.ShapeDtypeStruct((2, NPAD, NH), jnp.float32),
        ],
        mesh=_mesh,
        compiler_params=_params,
        scratch_types=[
            pltpu.VMEM((CHUNK,), jnp.int32),
            pltpu.VMEM((CHUNK,), jnp.int32),
            pltpu.VMEM((CHUNK, NH), jnp.float32),
            pltpu.VMEM((CHUNK, NH), jnp.float32),
            pltpu.VMEM((CHUNK, NH), jnp.float32),
            pltpu.VMEM((CHUNK,), jnp.int32),
            pltpu.VMEM((CHUNK,), jnp.int32),
            pltpu.VMEM((CHUNK, NH), jnp.float32),
            pltpu.VMEM((CHUNK, NH), jnp.float32),
            pltpu.VMEM((CHUNK, NH), jnp.float32),
            pltpu.VMEM((NPAD // 16, NH), jnp.float32),
            pltpu.VMEM_SHARED((NPAD, NH), jnp.float32),
            pltpu.SemaphoreType.DMA,
            pltpu.SemaphoreType.DMA,
            pltpu.SemaphoreType.DMA,
        ],
    )
    return f(src1d, trg1d, ss, st, zeros)


def _recip_body(parts_ref, r_ref):
    d = parts_ref[0] + parts_ref[1]
    r_ref[...] = 1.0 / (d + 1e-16)


def _recip(parts):
    return pl.pallas_call(
        _recip_body,
        out_shape=jax.ShapeDtypeStruct((NPAD, NH), jnp.float32),
    )(parts)


def _p2_body(trg_ref, exp_ref, rc_ref, out_ref,
             idx_t0, ev0, rv0, ov0,
             idx_t1, ev1, rv1, ov1,
             sem_i, sem_e, sem_g, sem_w):
    cid = lax.axis_index("c")
    sid = lax.axis_index("s")
    wid = sid * 2 + cid

    lane = lax.iota(jnp.int32, 16)
    bufs = ((idx_t0, ev0, rv0, ov0),
            (idx_t1, ev1, rv1, ov1))

    def eb(c):
        return wid * (NCHUNK * CHUNK) + c * CHUNK

    Is, Es, Gs = {}, {}, {}

    def start_i(c):
        bs = bufs[c & 1]
        Is[c] = pltpu.async_copy(trg_ref.at[pl.ds(eb(c), CHUNK)], bs[0], sem_i)
        Es[c] = pltpu.async_copy(exp_ref.at[pl.ds(eb(c), CHUNK)], bs[1], sem_e)

    def start_g(c):
        bs = bufs[c & 1]
        Gs[c] = pltpu.async_copy(rc_ref.at[bs[0]], bs[2], sem_g)

    start_i(0)
    start_i(1)
    Is[0].wait()
    start_g(0)

    for c in range(NCHUNK):
        bs = bufs[c & 1]
        Gs[c].wait()
        Es[c].wait()
        if c + 1 < NCHUNK:
            Is[c + 1].wait()
            start_g(c + 1)

        ev, rv, ov = bs[1], bs[2], bs[3]

        def vec(i, acc):
            pp = lane + 16 * i
            h = pp // CHUNK
            e = pp - h * CHUNK
            x = plsc.load_gather(ev, [e, h])
            r = plsc.load_gather(rv, [e, h])
            ov[pl.ds(16 * i, 16)] = x * r
            return acc

        lax.fori_loop(0, (CHUNK * NH) // 16, vec, 0, unroll=8)

        wr = []
        for h in range(NH):
            wr.append(pltpu.async_copy(
                ov.at[pl.ds(h * CHUNK, CHUNK)],
                out_ref.at[pl.ds(h * N_EDGES + eb(c), CHUNK)], sem_w))
        if c + 2 < NCHUNK:
            start_i(c + 2)
        for cp in wr:
            cp.wait()


def _pass2(trg1d, exp_e, recip):
    f = pl.kernel(
        _p2_body,
        out_type=jax.ShapeDtypeStruct((NH * N_EDGES,), jnp.float32),
        mesh=_mesh,
        compiler_params=_params,
        scratch_types=[
            pltpu.VMEM((CHUNK,), jnp.int32),
            pltpu.VMEM((CHUNK, NH), jnp.float32),
            pltpu.VMEM((CHUNK, NH), jnp.float32),
            pltpu.VMEM((CHUNK * NH,), jnp.float32),
            pltpu.VMEM((CHUNK,), jnp.int32),
            pltpu.VMEM((CHUNK, NH), jnp.float32),
            pltpu.VMEM((CHUNK, NH), jnp.float32),
            pltpu.VMEM((CHUNK * NH,), jnp.float32),
            pltpu.SemaphoreType.DMA,
            pltpu.SemaphoreType.DMA,
            pltpu.SemaphoreType.DMA,
            pltpu.SemaphoreType.DMA,
        ],
    )
    return f(trg1d, exp_e, recip)


def kernel(in_nodes_features, edge_index, linear_proj, scoring_fn_source,
           scoring_fn_target):
    x_pad = jnp.pad(in_nodes_features, ((0, NPAD - N_NODES), (0, 0)))
    asrc = scoring_fn_source.reshape(1, NH * F_OUT)
    atrg = scoring_fn_target.reshape(1, NH * F_OUT)
    ss, st = _scores(x_pad, linear_proj, asrc, atrg)

    src = edge_index[0]
    trg = edge_index[1]
    zeros = jnp.zeros((NPAD, NH), jnp.float32)

    exp_e, parts = _pass1(src, trg, ss, st, zeros)
    recip = _recip(parts)
    flat = _pass2(trg, exp_e, recip)
    return flat.reshape(NH, N_EDGES).transpose(1, 0).reshape(N_EDGES, NH, 1)


# speedup vs baseline: 15.3872x; 1.0155x over previous
import jax
import jax.numpy as jnp
from jax import lax
from jax.experimental import pallas as pl
from jax.experimental.pallas import tpu as pltpu
from jax.experimental.pallas import tpu_sc as plsc

N_NODES = 10000
N_EDGES = 320000
D_IN = 128
NH = 8
F_OUT = 16

NPAD = 10112
CHUNK = 1000
NCHUNK = N_EDGES // (32 * CHUNK)

_mesh = plsc.VectorSubcoreMesh(
    core_axis_name="c", subcore_axis_name="s", num_cores=2, num_subcores=16
)
_params = pltpu.CompilerParams(
    needs_layout_passes=False, use_tc_tiling_on_sc=False)


def _scores_body(x_ref, w_ref, asrc_ref, atrg_ref, ss_ref, st_ref):
    x = x_ref[...]
    w = w_ref[...]
    proj = jnp.dot(x, w, preferred_element_type=jnp.float32)
    col = lax.broadcasted_iota(jnp.int32, (D_IN, NH), 0)
    hd = lax.broadcasted_iota(jnp.int32, (D_IN, NH), 1)
    g = (col // F_OUT == hd).astype(jnp.float32)
    ss_ref[...] = jnp.dot(proj * asrc_ref[...], g, preferred_element_type=jnp.float32)
    st_ref[...] = jnp.dot(proj * atrg_ref[...], g, preferred_element_type=jnp.float32)


def _scores(x_pad, w, asrc, atrg):
    return pl.pallas_call(
        _scores_body,
        out_shape=[
            jax.ShapeDtypeStruct((NPAD, NH), jnp.float32),
            jax.ShapeDtypeStruct((NPAD, NH), jnp.float32),
        ],
    )(x_pad, w, asrc, atrg)


def _lane_patterns():
    lane = lax.iota(jnp.int32, 16)
    return jnp.right_shift(lane, 3), jnp.bitwise_and(lane, 7)


def _p1_body(src_ref, trg_ref, ss_ref, st_ref, z_ref, exp_ref, part_ref,
             idx_s0, idx_s1, idx_t0, idx_t1, idx_t2,
             rows_s0, rows_t0, exp_v0, rows_s1, rows_t1, exp_v1,
             stage_v, denom_sp, sem_i, sem_g, sem_w, sem_a):
    cid = lax.axis_index("c")
    sid = lax.axis_index("s")
    wid = sid * 2 + cid

    zrows = pl.ds(sid * (NPAD // 16), NPAD // 16)
    pltpu.sync_copy(z_ref.at[zrows], stage_v)
    pltpu.sync_copy(stage_v, denom_sp.at[zrows])
    plsc.subcore_barrier()

    prow, pcol = _lane_patterns()
    idx_s = (idx_s0, idx_s1)
    idx_t = (idx_t0, idx_t1, idx_t2)
    rows_s = (rows_s0, rows_s1)
    rows_t = (rows_t0, rows_t1)
    exp_v = (exp_v0, exp_v1)

    def eb(c):
        return wid * (NCHUNK * CHUNK) + c * CHUNK

    Is, Gs, As = {}, {}, {}

    def start_i(c):
        Is[c] = (
            pltpu.async_copy(src_ref.at[pl.ds(eb(c), CHUNK)], idx_s[c % 2], sem_i),
            pltpu.async_copy(trg_ref.at[pl.ds(eb(c), CHUNK)], idx_t[c % 3], sem_i),
        )

    def start_g(c):
        Gs[c] = (
            pltpu.async_copy(ss_ref.at[idx_s[c % 2]], rows_s[c % 2], sem_g),
            pltpu.async_copy(st_ref.at[idx_t[c % 3]], rows_t[c % 2], sem_g),
        )

    start_i(0)
    start_i(1)
    for cp in Is[0]:
        cp.wait()
    start_g(0)

    for c in range(NCHUNK):
        for cp in Gs[c]:
            cp.wait()
        if c + 1 < NCHUNK:
            for cp in Is[c + 1]:
                cp.wait()
            start_g(c + 1)

        rs, rt, ev = rows_s[c % 2], rows_t[c % 2], exp_v[c % 2]

        def vec(i, acc):
            rv = prow + 2 * i
            a = plsc.load_gather(rs, [rv, pcol])
            b = plsc.load_gather(rt, [rv, pcol])
            x = a + b
            wv = jnp.exp(jnp.maximum(x, x * 0.2))
            plsc.store_scatter(ev, [rv, pcol], wv)
            return acc

        lax.fori_loop(0, CHUNK // 2, vec, 0, unroll=8)

        wcp = pltpu.async_copy(ev, exp_ref.at[pl.ds(eb(c), CHUNK)], sem_w)
        if c > 0:
            As[c - 1].wait()
        As[c] = pltpu.async_copy(ev, denom_sp.at[idx_t[c % 3]], sem_a, add=True)
        if c + 2 < NCHUNK:
            start_i(c + 2)
        wcp.wait()

    As[NCHUNK - 1].wait()
    plsc.subcore_barrier()
    pltpu.sync_copy(denom_sp.at[zrows], stage_v)
    pltpu.sync_copy(stage_v, part_ref.at[cid, zrows])


def _pass1(src1d, trg1d, ss, st, zeros):
    f = pl.kernel(
        _p1_body,
        out_type=[
            jax.ShapeDtypeStruct((N_EDGES, NH), jnp.float32),
            jax.ShapeDtypeStruct((2, NPAD, NH), jnp.float32),
        ],
        mesh=_mesh,
        compiler_params=_params,
        scratch_types=[
            pltpu.VMEM((CHUNK,), jnp.int32),
            pltpu.VMEM((CHUNK,), jnp.int32),
            pltpu.VMEM((CHUNK,), jnp.int32),
            pltpu.VMEM((CHUNK,), jnp.int32),
            pltpu.VMEM((CHUNK,), jnp.int32),
            pltpu.VMEM((CHUNK, NH), jnp.float32),
            pltpu.VMEM((CHUNK, NH), jnp.float32),
            pltpu.VMEM((CHUNK, NH), jnp.float32),
            pltpu.VMEM((CHUNK, NH), jnp.float32),
            pltpu.VMEM((CHUNK, NH), jnp.float32),
            pltpu.VMEM((CHUNK, NH), jnp.float32),
            pltpu.VMEM((NPAD // 16, NH), jnp.float32),
            pltpu.VMEM_SHARED((NPAD, NH), jnp.float32),
            pltpu.SemaphoreType.DMA,
            pltpu.SemaphoreType.DMA,
            pltpu.SemaphoreType.DMA,
            pltpu.SemaphoreType.DMA,
        ],
    )
    return f(src1d, trg1d, ss, st, zeros)


def _recip_body(parts_ref, r_ref):
    d = parts_ref[0] + parts_ref[1]
    r_ref[...] = 1.0 / (d + 1e-16)


def _recip(parts):
    return pl.pallas_call(
        _recip_body,
        out_shape=jax.ShapeDtypeStruct((NPAD, NH), jnp.float32),
    )(parts)


def _p2_body(trg_ref, exp_ref, rc_ref, out_ref,
             idx_t0, ev0, rv0, ov0,
             idx_t1, ev1, rv1, ov1,
             sem_i, sem_e, sem_g, sem_w):
    cid = lax.axis_index("c")
    sid = lax.axis_index("s")
    wid = sid * 2 + cid

    lane = lax.iota(jnp.int32, 16)
    bufs = ((idx_t0, ev0, rv0, ov0),
            (idx_t1, ev1, rv1, ov1))

    def eb(c):
        return wid * (NCHUNK * CHUNK) + c * CHUNK

    Is, Es, Gs = {}, {}, {}

    def start_i(c):
        bs = bufs[c & 1]
        Is[c] = pltpu.async_copy(trg_ref.at[pl.ds(eb(c), CHUNK)], bs[0], sem_i)
        Es[c] = pltpu.async_copy(exp_ref.at[pl.ds(eb(c), CHUNK)], bs[1], sem_e)

    def start_g(c):
        bs = bufs[c & 1]
        Gs[c] = pltpu.async_copy(rc_ref.at[bs[0]], bs[2], sem_g)

    start_i(0)
    start_i(1)
    Is[0].wait()
    start_g(0)

    for c in range(NCHUNK):
        bs = bufs[c & 1]
        Gs[c].wait()
        Es[c].wait()
        if c + 1 < NCHUNK:
            Is[c + 1].wait()
            start_g(c + 1)

        ev, rv, ov = bs[1], bs[2], bs[3]

        def vec(i, acc):
            pp = lane + 16 * i
            h = pp // CHUNK
            e = pp - h * CHUNK
            x = plsc.load_gather(ev, [e, h])
            r = plsc.load_gather(rv, [e, h])
            ov[pl.ds(16 * i, 16)] = x * r
            return acc

        lax.fori_loop(0, (CHUNK * NH) // 16, vec, 0, unroll=8)

        wr = []
        for h in range(NH):
            wr.append(pltpu.async_copy(
                ov.at[pl.ds(h * CHUNK, CHUNK)],
                out_ref.at[pl.ds(h * N_EDGES + eb(c), CHUNK)], sem_w))
        if c + 2 < NCHUNK:
            start_i(c + 2)
        for cp in wr:
            cp.wait()


def _pass2(trg1d, exp_e, recip):
    f = pl.kernel(
        _p2_body,
        out_type=jax.ShapeDtypeStruct((NH * N_EDGES,), jnp.float32),
        mesh=_mesh,
        compiler_params=_params,
        scratch_types=[
            pltpu.VMEM((CHUNK,), jnp.int32),
            pltpu.VMEM((CHUNK, NH), jnp.float32),
            pltpu.VMEM((CHUNK, NH), jnp.float32),
            pltpu.VMEM((CHUNK * NH,), jnp.float32),
            pltpu.VMEM((CHUNK,), jnp.int32),
            pltpu.VMEM((CHUNK, NH), jnp.float32),
            pltpu.VMEM((CHUNK, NH), jnp.float32),
            pltpu.VMEM((CHUNK * NH,), jnp.float32),
            pltpu.SemaphoreType.DMA,
            pltpu.SemaphoreType.DMA,
            pltpu.SemaphoreType.DMA,
            pltpu.SemaphoreType.DMA,
        ],
    )
    return f(trg1d, exp_e, recip)


def kernel(in_nodes_features, edge_index, linear_proj, scoring_fn_source,
           scoring_fn_target):
    x_pad = jnp.pad(in_nodes_features, ((0, NPAD - N_NODES), (0, 0)))
    asrc = scoring_fn_source.reshape(1, NH * F_OUT)
    atrg = scoring_fn_target.reshape(1, NH * F_OUT)
    ss, st = _scores(x_pad, linear_proj, asrc, atrg)

    src = edge_index[0]
    trg = edge_index[1]
    zeros = jnp.zeros((NPAD, NH), jnp.float32)

    exp_e, parts = _pass1(src, trg, ss, st, zeros)
    recip = _recip(parts)
    flat = _pass2(trg, exp_e, recip)
    return flat.reshape(NH, N_EDGES).transpose(1, 0).reshape(N_EDGES, NH, 1)


# speedup vs baseline: 15.6833x; 1.0192x over previous
import jax
import jax.numpy as jnp
from jax import lax
from jax.experimental import pallas as pl
from jax.experimental.pallas import tpu as pltpu
from jax.experimental.pallas import tpu_sc as plsc

N_NODES = 10000
N_EDGES = 320000
D_IN = 128
NH = 8
F_OUT = 16

NPAD = 10112
CHUNK = 2000
NCHUNK = N_EDGES // (32 * CHUNK)

_mesh = plsc.VectorSubcoreMesh(
    core_axis_name="c", subcore_axis_name="s", num_cores=2, num_subcores=16
)
_params = pltpu.CompilerParams(
    needs_layout_passes=False, use_tc_tiling_on_sc=False)


def _scores_body(x_ref, w_ref, asrc_ref, atrg_ref, ss_ref, st_ref):
    x = x_ref[...]
    w = w_ref[...]
    proj = jnp.dot(x, w, preferred_element_type=jnp.float32)
    col = lax.broadcasted_iota(jnp.int32, (D_IN, NH), 0)
    hd = lax.broadcasted_iota(jnp.int32, (D_IN, NH), 1)
    g = (col // F_OUT == hd).astype(jnp.float32)
    ss_ref[...] = jnp.dot(proj * asrc_ref[...], g, preferred_element_type=jnp.float32)
    st_ref[...] = jnp.dot(proj * atrg_ref[...], g, preferred_element_type=jnp.float32)


def _scores(x_pad, w, asrc, atrg):
    return pl.pallas_call(
        _scores_body,
        out_shape=[
            jax.ShapeDtypeStruct((NPAD, NH), jnp.float32),
            jax.ShapeDtypeStruct((NPAD, NH), jnp.float32),
        ],
    )(x_pad, w, asrc, atrg)


def _lane_patterns():
    lane = lax.iota(jnp.int32, 16)
    return jnp.right_shift(lane, 3), jnp.bitwise_and(lane, 7)


def _p1_body(src_ref, trg_ref, ss_ref, st_ref, z_ref, exp_ref, part_ref,
             idx_s0, idx_s1, idx_t0, idx_t1, idx_t2,
             rows_s0, rows_t0, exp_v0, rows_s1, rows_t1, exp_v1,
             stage_v, denom_sp, sem_i, sem_g, sem_w, sem_a):
    cid = lax.axis_index("c")
    sid = lax.axis_index("s")
    wid = sid * 2 + cid

    zrows = pl.ds(sid * (NPAD // 16), NPAD // 16)
    pltpu.sync_copy(z_ref.at[zrows], stage_v)
    pltpu.sync_copy(stage_v, denom_sp.at[zrows])
    plsc.subcore_barrier()

    prow, pcol = _lane_patterns()
    idx_s = (idx_s0, idx_s1)
    idx_t = (idx_t0, idx_t1, idx_t2)
    rows_s = (rows_s0, rows_s1)
    rows_t = (rows_t0, rows_t1)
    exp_v = (exp_v0, exp_v1)

    def eb(c):
        return wid * (NCHUNK * CHUNK) + c * CHUNK

    Is, Gs, As = {}, {}, {}

    def start_i(c):
        Is[c] = (
            pltpu.async_copy(src_ref.at[pl.ds(eb(c), CHUNK)], idx_s[c % 2], sem_i),
            pltpu.async_copy(trg_ref.at[pl.ds(eb(c), CHUNK)], idx_t[c % 3], sem_i),
        )

    def start_g(c):
        Gs[c] = (
            pltpu.async_copy(ss_ref.at[idx_s[c % 2]], rows_s[c % 2], sem_g),
            pltpu.async_copy(st_ref.at[idx_t[c % 3]], rows_t[c % 2], sem_g),
        )

    start_i(0)
    start_i(1)
    for cp in Is[0]:
        cp.wait()
    start_g(0)

    for c in range(NCHUNK):
        for cp in Gs[c]:
            cp.wait()
        if c + 1 < NCHUNK:
            for cp in Is[c + 1]:
                cp.wait()
            start_g(c + 1)

        rs, rt, ev = rows_s[c % 2], rows_t[c % 2], exp_v[c % 2]

        def vec(i, acc):
            rv = prow + 2 * i
            a = plsc.load_gather(rs, [rv, pcol])
            b = plsc.load_gather(rt, [rv, pcol])
            x = a + b
            wv = jnp.exp(jnp.maximum(x, x * 0.2))
            plsc.store_scatter(ev, [rv, pcol], wv)
            return acc

        lax.fori_loop(0, CHUNK // 2, vec, 0, unroll=8)

        wcp = pltpu.async_copy(ev, exp_ref.at[pl.ds(eb(c), CHUNK)], sem_w)
        if c > 0:
            As[c - 1].wait()
        As[c] = pltpu.async_copy(ev, denom_sp.at[idx_t[c % 3]], sem_a, add=True)
        if c + 2 < NCHUNK:
            start_i(c + 2)
        wcp.wait()

    As[NCHUNK - 1].wait()
    plsc.subcore_barrier()
    pltpu.sync_copy(denom_sp.at[zrows], stage_v)
    pltpu.sync_copy(stage_v, part_ref.at[cid, zrows])


def _pass1(src1d, trg1d, ss, st, zeros):
    f = pl.kernel(
        _p1_body,
        out_type=[
            jax.ShapeDtypeStruct((N_EDGES, NH), jnp.float32),
            jax.ShapeDtypeStruct((2, NPAD, NH), jnp.float32),
        ],
        mesh=_mesh,
        compiler_params=_params,
        scratch_types=[
            pltpu.VMEM((CHUNK,), jnp.int32),
            pltpu.VMEM((CHUNK,), jnp.int32),
            pltpu.VMEM((CHUNK,), jnp.int32),
            pltpu.VMEM((CHUNK,), jnp.int32),
            pltpu.VMEM((CHUNK,), jnp.int32),
            pltpu.VMEM((CHUNK, NH), jnp.float32),
            pltpu.VMEM((CHUNK, NH), jnp.float32),
            pltpu.VMEM((CHUNK, NH), jnp.float32),
            pltpu.VMEM((CHUNK, NH), jnp.float32),
            pltpu.VMEM((CHUNK, NH), jnp.float32),
            pltpu.VMEM((CHUNK, NH), jnp.float32),
            pltpu.VMEM((NPAD // 16, NH), jnp.float32),
            pltpu.VMEM_SHARED((NPAD, NH), jnp.float32),
            pltpu.SemaphoreType.DMA,
            pltpu.SemaphoreType.DMA,
            pltpu.SemaphoreType.DMA,
            pltpu.SemaphoreType.DMA,
        ],
    )
    return f(src1d, trg1d, ss, st, zeros)


def _recip_body(parts_ref, r_ref):
    d = parts_ref[0] + parts_ref[1]
    r_ref[...] = 1.0 / (d + 1e-16)


def _recip(parts):
    return pl.pallas_call(
        _recip_body,
        out_shape=jax.ShapeDtypeStruct((NPAD, NH), jnp.float32),
    )(parts)


def _p2_body(trg_ref, exp_ref, rc_ref, out_ref,
             idx_t0, ev0, rv0, ov0,
             idx_t1, ev1, rv1, ov1,
             sem_i, sem_e, sem_g, sem_w):
    cid = lax.axis_index("c")
    sid = lax.axis_index("s")
    wid = sid * 2 + cid

    lane = lax.iota(jnp.int32, 16)
    bufs = ((idx_t0, ev0, rv0, ov0),
            (idx_t1, ev1, rv1, ov1))

    def eb(c):
        return wid * (NCHUNK * CHUNK) + c * CHUNK

    Is, Es, Gs = {}, {}, {}

    def start_i(c):
        bs = bufs[c & 1]
        Is[c] = pltpu.async_copy(trg_ref.at[pl.ds(eb(c), CHUNK)], bs[0], sem_i)
        Es[c] = pltpu.async_copy(exp_ref.at[pl.ds(eb(c), CHUNK)], bs[1], sem_e)

    def start_g(c):
        bs = bufs[c & 1]
        Gs[c] = pltpu.async_copy(rc_ref.at[bs[0]], bs[2], sem_g)

    start_i(0)
    start_i(1)
    Is[0].wait()
    start_g(0)

    for c in range(NCHUNK):
        bs = bufs[c & 1]
        Gs[c].wait()
        Es[c].wait()
        if c + 1 < NCHUNK:
            Is[c + 1].wait()
            start_g(c + 1)

        ev, rv, ov = bs[1], bs[2], bs[3]

        def vec(i, acc):
            pp = lane + 16 * i
            h = pp // CHUNK
            e = pp - h * CHUNK
            x = plsc.load_gather(ev, [e, h])
            r = plsc.load_gather(rv, [e, h])
            ov[pl.ds(16 * i, 16)] = x * r
            return acc

        lax.fori_loop(0, (CHUNK * NH) // 16, vec, 0, unroll=8)

        wr = []
        for h in range(NH):
            wr.append(pltpu.async_copy(
                ov.at[pl.ds(h * CHUNK, CHUNK)],
                out_ref.at[pl.ds(h * N_EDGES + eb(c), CHUNK)], sem_w))
        if c + 2 < NCHUNK:
            start_i(c + 2)
        for cp in wr:
            cp.wait()


def _pass2(trg1d, exp_e, recip):
    f = pl.kernel(
        _p2_body,
        out_type=jax.ShapeDtypeStruct((NH * N_EDGES,), jnp.float32),
        mesh=_mesh,
        compiler_params=_params,
        scratch_types=[
            pltpu.VMEM((CHUNK,), jnp.int32),
            pltpu.VMEM((CHUNK, NH), jnp.float32),
            pltpu.VMEM((CHUNK, NH), jnp.float32),
            pltpu.VMEM((CHUNK * NH,), jnp.float32),
            pltpu.VMEM((CHUNK,), jnp.int32),
            pltpu.VMEM((CHUNK, NH), jnp.float32),
            pltpu.VMEM((CHUNK, NH), jnp.float32),
            pltpu.VMEM((CHUNK * NH,), jnp.float32),
            pltpu.SemaphoreType.DMA,
            pltpu.SemaphoreType.DMA,
            pltpu.SemaphoreType.DMA,
            pltpu.SemaphoreType.DMA,
        ],
    )
    return f(trg1d, exp_e, recip)


def kernel(in_nodes_features, edge_index, linear_proj, scoring_fn_source,
           scoring_fn_target):
    x_pad = jnp.pad(in_nodes_features, ((0, NPAD - N_NODES), (0, 0)))
    asrc = scoring_fn_source.reshape(1, NH * F_OUT)
    atrg = scoring_fn_target.reshape(1, NH * F_OUT)
    ss, st = _scores(x_pad, linear_proj, asrc, atrg)

    src = edge_index[0]
    trg = edge_index[1]
    zeros = jnp.zeros((NPAD, NH), jnp.float32)

    exp_e, parts = _pass1(src, trg, ss, st, zeros)
    recip = _recip(parts)
    flat = _pass2(trg, exp_e, recip)
    return flat.reshape(NH, N_EDGES).transpose(1, 0).reshape(N_EDGES, NH, 1)
